# baseline (speedup 1.0000x reference)
import functools

import jax
import jax.numpy as jnp
from jax import lax
from jax.experimental import pallas as pl
from jax.experimental.pallas import tpu as pltpu
from jax.experimental.pallas import tpu_sc as plsc

EMB = 64
B = 16384

NC = 2
NS = 16
NW = NC * NS
R = B // NW
GCHUNK = 128
CCHUNK = 16


def _body(user_hbm, item_hbm, loc_hbm, age_hbm, wf_hbm, wa_hbm, ba_hbm,
          ut_hbm, it_hbm, lt_hbm, out_hbm,
          idx_u, idx_i, idx_l, age_v, u_rows, i_rows, l_rows,
          acc_buf, wf_v, wa_v, ba_v, out_v, sem):
    wid = lax.axis_index("s") * NC + lax.axis_index("c")
    base = wid * R

    pltpu.sync_copy(user_hbm.at[pl.ds(base, R)], idx_u)
    pltpu.sync_copy(item_hbm.at[pl.ds(base, R)], idx_i)
    pltpu.sync_copy(loc_hbm.at[pl.ds(base, R)], idx_l)
    pltpu.sync_copy(age_hbm.at[pl.ds(base, R)], age_v)
    pltpu.sync_copy(wf_hbm, wf_v)
    pltpu.sync_copy(wa_hbm, wa_v)
    pltpu.sync_copy(ba_hbm, ba_v)

    copies = []
    for k in range(R // GCHUNK):
        sl = pl.ds(k * GCHUNK, GCHUNK)
        copies.append(pltpu.async_copy(ut_hbm.at[idx_u.at[sl]], u_rows.at[sl], sem))
        copies.append(pltpu.async_copy(it_hbm.at[idx_i.at[sl]], i_rows.at[sl], sem))
        copies.append(pltpu.async_copy(lt_hbm.at[idx_l.at[sl]], l_rows.at[sl], sem))
    for c in copies:
        c.wait()

    wfv = wf_v[...]
    wav = wa_v[...]
    bav = ba_v[...]
    w_sim = wfv[0]
    c1 = (wfv[9] * wav[0] + wfv[10] * wav[1]
          + wfv[11] * wav[2] + wfv[12] * wav[3])
    c2 = (wfv[9] * bav[0] + wfv[10] * bav[1]
          + wfv[11] * bav[2] + wfv[12] * bav[3] + wfv[13])
    lane = lax.iota(jnp.int32, CCHUNK)

    def chunk_body(c, carry):
        r0 = pl.multiple_of(c * CCHUNK, CCHUNK)
        for rr in range(CCHUNK):
            r = r0 + rr
            acc = u_rows[r, pl.ds(0, 16)] * i_rows[r, pl.ds(0, 16)]
            acc = acc + u_rows[r, pl.ds(16, 16)] * i_rows[r, pl.ds(16, 16)]
            acc = acc + u_rows[r, pl.ds(32, 16)] * i_rows[r, pl.ds(32, 16)]
            acc = acc + u_rows[r, pl.ds(48, 16)] * i_rows[r, pl.ds(48, 16)]
            acc_buf[rr, :] = acc
        sim = plsc.load_gather(acc_buf, [lane, jnp.zeros((16,), jnp.int32)])
        for j in range(1, 16):
            sim = sim + plsc.load_gather(acc_buf, [lane, jnp.full((16,), j, jnp.int32)])
        rows16 = r0 + lane
        loc_acc = jnp.zeros((16,), jnp.float32)
        for j in range(8):
            lv = plsc.load_gather(l_rows, [rows16, jnp.full((16,), j, jnp.int32)])
            loc_acc = loc_acc + lv * wfv[1 + j]
        a16 = age_v[pl.ds(r0, 16)]
        out_v[pl.ds(r0, 16)] = sim * w_sim + loc_acc + a16 * c1 + c2
        return carry

    lax.fori_loop(0, R // CCHUNK, chunk_body, 0)
    pltpu.sync_copy(out_v, out_hbm.at[pl.ds(base, R)])


_ncf = functools.partial(
    pl.kernel,
    mesh=plsc.VectorSubcoreMesh(core_axis_name="c", subcore_axis_name="s"),
    out_type=jax.ShapeDtypeStruct((B,), jnp.float32),
    compiler_params=pltpu.CompilerParams(
        needs_layout_passes=False, use_tc_tiling_on_sc=False),
    scratch_types=[
        pltpu.VMEM((R,), jnp.int32),
        pltpu.VMEM((R,), jnp.int32),
        pltpu.VMEM((R,), jnp.int32),
        pltpu.VMEM((R,), jnp.float32),
        pltpu.VMEM((R, EMB), jnp.float32),
        pltpu.VMEM((R, EMB), jnp.float32),
        pltpu.VMEM((R, 8), jnp.float32),
        pltpu.VMEM((CCHUNK, 16), jnp.float32),
        pltpu.VMEM((16,), jnp.float32),
        pltpu.VMEM((16,), jnp.float32),
        pltpu.VMEM((16,), jnp.float32),
        pltpu.VMEM((R,), jnp.float32),
        pltpu.SemaphoreType.DMA,
    ],
)(_body)


def kernel(user, item, location, age, user_table, item_table, loc_table,
           W_age, b_age, W_final, b_final):
    wf = jnp.concatenate([W_final[0], b_final, jnp.zeros((2,), jnp.float32)])
    wa = jnp.concatenate([W_age[:, 0], jnp.zeros((12,), jnp.float32)])
    ba = jnp.concatenate([b_age, jnp.zeros((12,), jnp.float32)])
    return _ncf(user, item, location, age.astype(jnp.float32), wf, wa, ba,
                user_table, item_table, loc_table)
